# baseline (speedup 1.0000x reference)
import functools

import jax
import jax.numpy as jnp
from jax import lax
from jax.experimental import pallas as pl
from jax.experimental.pallas import tpu as pltpu
from jax.experimental.pallas import tpu_sc as plsc

_POOL_SIZE = 1000
_LENGTH = 4
_EMBED_DIM = 128
_BATCH = 16384
_TOP_K = 5

_D = _LENGTH * _EMBED_DIM
_B_TOTAL = _BATCH * _TOP_K
_NC, _NS = 1, 16
_NW = _NC * _NS
_BATCH_PER_W = _BATCH // _NW
_CB = 16
_CHUNK = _CB * _TOP_K
_N_CHUNKS = _BATCH_PER_W // _CB

_mesh = plsc.VectorSubcoreMesh(
    core_axis_name="c", subcore_axis_name="s",
    num_cores=_NC, num_subcores=_NS,
)


_NBUF = 2


@functools.partial(
    pl.kernel,
    out_type=jax.ShapeDtypeStruct((_BATCH, _TOP_K * _LENGTH, _EMBED_DIM), jnp.float32),
    mesh=_mesh,
    scratch_types=[
        pltpu.VMEM((_N_CHUNKS, _CHUNK), jnp.int32),
        [pltpu.VMEM((_CHUNK, _LENGTH, _EMBED_DIM), jnp.float32) for _ in range(_NBUF)],
        [pltpu.SemaphoreType.DMA for _ in range(_NBUF)],
    ],
)
def _gather_rows(idx_hbm, table_hbm, out_hbm, idx_v, bufs, sems):
    wid = lax.axis_index("s") * _NC + lax.axis_index("c")
    pltpu.sync_copy(idx_hbm.at[wid], idx_v)
    base = wid * _BATCH_PER_W

    def write_out(b, c):
        src = bufs[b].reshape(_CB, _TOP_K * _LENGTH, _EMBED_DIM)
        pltpu.sync_copy(src, out_hbm.at[pl.ds(base + c * _CB, _CB)])

    for b in range(_NBUF):
        pltpu.async_copy(table_hbm.at[idx_v.at[b]], bufs[b], sems[b])

    @pl.loop(0, _N_CHUNKS - _NBUF, step=_NBUF)
    def _(g):
        for b in range(_NBUF):
            c = g + b
            pltpu.make_async_copy(table_hbm.at[idx_v.at[c]], bufs[b], sems[b]).wait()
            write_out(b, c)
            pltpu.async_copy(table_hbm.at[idx_v.at[c + _NBUF]], bufs[b], sems[b])

    for b in range(_NBUF):
        c = _N_CHUNKS - _NBUF + b
        pltpu.make_async_copy(table_hbm.at[idx_v.at[c]], bufs[b], sems[b]).wait()
        write_out(b, c)


def kernel(prompt_mask, prompt):
    idx = prompt_mask.astype(jnp.int32).reshape(_NW, _N_CHUNKS, _CHUNK)
    return _gather_rows(idx, prompt)


# speedup vs baseline: 1.3865x; 1.3865x over previous
import functools

import jax
import jax.numpy as jnp
from jax import lax
from jax.experimental import pallas as pl
from jax.experimental.pallas import tpu as pltpu
from jax.experimental.pallas import tpu_sc as plsc

_POOL_SIZE = 1000
_LENGTH = 4
_EMBED_DIM = 128
_BATCH = 16384
_TOP_K = 5

_D = _LENGTH * _EMBED_DIM
_B_TOTAL = _BATCH * _TOP_K
_NC, _NS = 2, 16
_NW = _NC * _NS
_BATCH_PER_W = _BATCH // _NW
_CB = 16
_CHUNK = _CB * _TOP_K
_N_CHUNKS = _BATCH_PER_W // _CB

_mesh = plsc.VectorSubcoreMesh(
    core_axis_name="c", subcore_axis_name="s",
    num_cores=_NC, num_subcores=_NS,
)


_NBUF = 2


@functools.partial(
    pl.kernel,
    out_type=jax.ShapeDtypeStruct((_BATCH, _TOP_K * _LENGTH, _EMBED_DIM), jnp.float32),
    mesh=_mesh,
    scratch_types=[
        pltpu.VMEM((_N_CHUNKS, _CHUNK), jnp.int32),
        [pltpu.VMEM((_CHUNK, _LENGTH, _EMBED_DIM), jnp.float32) for _ in range(_NBUF)],
        [pltpu.SemaphoreType.DMA for _ in range(_NBUF)],
        pltpu.VMEM_SHARED((_POOL_SIZE, _LENGTH, _EMBED_DIM), jnp.float32),
    ],
)
def _gather_rows(idx_hbm, table_hbm, out_hbm, idx_v, bufs, sems, table_sp):
    sid = lax.axis_index("s")
    wid = sid * _NC + lax.axis_index("c")
    pltpu.sync_copy(idx_hbm.at[wid], idx_v)
    base = wid * _BATCH_PER_W

    @pl.when(sid < 8)
    def _():
        pltpu.sync_copy(
            table_hbm.at[pl.ds(sid * 125, 125)], table_sp.at[pl.ds(sid * 125, 125)]
        )
    plsc.subcore_barrier()

    def write_out(b, c):
        src = bufs[b].reshape(_CB, _TOP_K * _LENGTH, _EMBED_DIM)
        pltpu.sync_copy(src, out_hbm.at[pl.ds(base + c * _CB, _CB)])

    for b in range(_NBUF):
        pltpu.async_copy(table_sp.at[idx_v.at[b]], bufs[b], sems[b])

    @pl.loop(0, _N_CHUNKS - _NBUF, step=_NBUF)
    def _(g):
        for b in range(_NBUF):
            c = g + b
            pltpu.make_async_copy(table_sp.at[idx_v.at[c]], bufs[b], sems[b]).wait()
            write_out(b, c)
            pltpu.async_copy(table_sp.at[idx_v.at[c + _NBUF]], bufs[b], sems[b])

    for b in range(_NBUF):
        c = _N_CHUNKS - _NBUF + b
        pltpu.make_async_copy(table_sp.at[idx_v.at[c]], bufs[b], sems[b]).wait()
        write_out(b, c)


def kernel(prompt_mask, prompt):
    idx = prompt_mask.astype(jnp.int32).reshape(_NW, _N_CHUNKS, _CHUNK)
    return _gather_rows(idx, prompt)


# speedup vs baseline: 3.4179x; 2.4652x over previous
import functools

import jax
import jax.numpy as jnp
from jax import lax
from jax.experimental import pallas as pl
from jax.experimental.pallas import tpu as pltpu
from jax.experimental.pallas import tpu_sc as plsc

_POOL_SIZE = 1000
_LENGTH = 4
_EMBED_DIM = 128
_BATCH = 16384
_TOP_K = 5

_J = _TOP_K * _LENGTH
_ROWS = _POOL_SIZE * _LENGTH
_N_TOTAL = _BATCH * _J
_NC, _NS = 2, 16
_NW = _NC * _NS
_PER_W = _N_TOTAL // _NW
_CHUNK = 128
_N_CHUNKS = _PER_W // _CHUNK
_STAGE = 256

_mesh = plsc.VectorSubcoreMesh(
    core_axis_name="c", subcore_axis_name="s",
    num_cores=_NC, num_subcores=_NS,
)

_NBUF = 2


@functools.partial(
    pl.kernel,
    out_type=jax.ShapeDtypeStruct((_J, _BATCH, _EMBED_DIM), jnp.float32),
    mesh=_mesh,
    scratch_types=[
        pltpu.VMEM((_N_CHUNKS, _CHUNK), jnp.int32),
        [pltpu.VMEM((_CHUNK, _EMBED_DIM), jnp.float32) for _ in range(_NBUF)],
        [pltpu.SemaphoreType.DMA for _ in range(_NBUF)],
        pltpu.VMEM_SHARED((_ROWS, _EMBED_DIM), jnp.float32),
    ],
)
def _gather_rows(idx_hbm, table_hbm, out_hbm, idx_v, bufs, sems, table_sp):
    sid = lax.axis_index("s")
    wid = sid * _NC + lax.axis_index("c")
    pltpu.sync_copy(idx_hbm.at[wid], idx_v)
    base = wid * _PER_W
    out_flat = out_hbm.reshape(_N_TOTAL, _EMBED_DIM)

    @pl.when(sid < _NS - 1)
    def _():
        pltpu.sync_copy(
            table_hbm.at[pl.ds(sid * _STAGE, _STAGE)],
            table_sp.at[pl.ds(sid * _STAGE, _STAGE)],
        )

    @pl.when(sid == _NS - 1)
    def _():
        pltpu.sync_copy(
            table_hbm.at[pl.ds((_NS - 1) * _STAGE, _ROWS - (_NS - 1) * _STAGE)],
            table_sp.at[pl.ds((_NS - 1) * _STAGE, _ROWS - (_NS - 1) * _STAGE)],
        )

    plsc.subcore_barrier()

    for b in range(_NBUF):
        pltpu.async_copy(table_sp.at[idx_v.at[b]], bufs[b], sems[b])

    @pl.loop(0, _N_CHUNKS - _NBUF, step=_NBUF)
    def _(g):
        for b in range(_NBUF):
            c = g + b
            pltpu.make_async_copy(table_sp.at[idx_v.at[c]], bufs[b], sems[b]).wait()
            pltpu.sync_copy(bufs[b], out_flat.at[pl.ds(base + c * _CHUNK, _CHUNK)])
            pltpu.async_copy(table_sp.at[idx_v.at[c + _NBUF]], bufs[b], sems[b])

    for b in range(_NBUF):
        c = _N_CHUNKS - _NBUF + b
        pltpu.make_async_copy(table_sp.at[idx_v.at[c]], bufs[b], sems[b]).wait()
        pltpu.sync_copy(bufs[b], out_flat.at[pl.ds(base + c * _CHUNK, _CHUNK)])


def kernel(prompt_mask, prompt):
    idx = prompt_mask.astype(jnp.int32)
    sub = jnp.arange(_LENGTH, dtype=jnp.int32)
    eidx = idx[:, :, None] * _LENGTH + sub[None, None, :]
    eidx_w = eidx.reshape(_BATCH, _J).T.reshape(_NW, _N_CHUNKS, _CHUNK)
    table = prompt.reshape(_ROWS, _EMBED_DIM)
    out_t = _gather_rows(eidx_w, table)
    return out_t.transpose(1, 0, 2)


# speedup vs baseline: 3.4787x; 1.0178x over previous
import functools

import jax
import jax.numpy as jnp
from jax import lax
from jax.experimental import pallas as pl
from jax.experimental.pallas import tpu as pltpu
from jax.experimental.pallas import tpu_sc as plsc

_POOL_SIZE = 1000
_LENGTH = 4
_EMBED_DIM = 128
_BATCH = 16384
_TOP_K = 5

_J = _TOP_K * _LENGTH
_ROWS = _POOL_SIZE * _LENGTH
_N_TOTAL = _BATCH * _J
_NC, _NS = 2, 16
_NW = _NC * _NS
_PER_W = _N_TOTAL // _NW
_CHUNK = 128
_N_CHUNKS = _PER_W // _CHUNK
_STAGE = 256

_mesh = plsc.VectorSubcoreMesh(
    core_axis_name="c", subcore_axis_name="s",
    num_cores=_NC, num_subcores=_NS,
)

_NBUF = 4


@functools.partial(
    pl.kernel,
    out_type=jax.ShapeDtypeStruct((_J, _BATCH, _EMBED_DIM), jnp.float32),
    mesh=_mesh,
    scratch_types=[
        pltpu.VMEM((_N_CHUNKS, _CHUNK), jnp.int32),
        [pltpu.VMEM((_CHUNK, _EMBED_DIM), jnp.float32) for _ in range(_NBUF)],
        [pltpu.SemaphoreType.DMA for _ in range(_NBUF)],
        pltpu.VMEM_SHARED((_ROWS, _EMBED_DIM), jnp.float32),
    ],
)
def _gather_rows(idx_hbm, table_hbm, out_hbm, idx_v, bufs, sems, table_sp):
    sid = lax.axis_index("s")
    wid = sid * _NC + lax.axis_index("c")
    pltpu.sync_copy(idx_hbm.at[wid], idx_v)
    base = wid * _PER_W
    out_flat = out_hbm.reshape(_N_TOTAL, _EMBED_DIM)

    @pl.when(sid < _NS - 1)
    def _():
        pltpu.sync_copy(
            table_hbm.at[pl.ds(sid * _STAGE, _STAGE)],
            table_sp.at[pl.ds(sid * _STAGE, _STAGE)],
        )

    @pl.when(sid == _NS - 1)
    def _():
        pltpu.sync_copy(
            table_hbm.at[pl.ds((_NS - 1) * _STAGE, _ROWS - (_NS - 1) * _STAGE)],
            table_sp.at[pl.ds((_NS - 1) * _STAGE, _ROWS - (_NS - 1) * _STAGE)],
        )

    plsc.subcore_barrier()

    for b in range(_NBUF):
        pltpu.async_copy(table_sp.at[idx_v.at[b]], bufs[b], sems[b])

    @pl.loop(0, _N_CHUNKS - _NBUF, step=_NBUF)
    def _(g):
        for b in range(_NBUF):
            c = g + b
            pltpu.make_async_copy(table_sp.at[idx_v.at[c]], bufs[b], sems[b]).wait()
            pltpu.sync_copy(bufs[b], out_flat.at[pl.ds(base + c * _CHUNK, _CHUNK)])
            pltpu.async_copy(table_sp.at[idx_v.at[c + _NBUF]], bufs[b], sems[b])

    for b in range(_NBUF):
        c = _N_CHUNKS - _NBUF + b
        pltpu.make_async_copy(table_sp.at[idx_v.at[c]], bufs[b], sems[b]).wait()
        pltpu.sync_copy(bufs[b], out_flat.at[pl.ds(base + c * _CHUNK, _CHUNK)])


def kernel(prompt_mask, prompt):
    idx = prompt_mask.astype(jnp.int32)
    sub = jnp.arange(_LENGTH, dtype=jnp.int32)
    eidx = idx[:, :, None] * _LENGTH + sub[None, None, :]
    eidx_w = eidx.reshape(_BATCH, _J).T.reshape(_NW, _N_CHUNKS, _CHUNK)
    table = prompt.reshape(_ROWS, _EMBED_DIM)
    out_t = _gather_rows(eidx_w, table)
    return out_t.transpose(1, 0, 2)
